# baseline (speedup 1.0000x reference)
import jax
import jax.numpy as jnp
from jax.experimental import pallas as pl
from jax.experimental.pallas import tpu as pltpu

N_USERS = 10000
N_ITEMS = 10000
N_GROUPS = 2048
D = 512

BG = 256
NG = N_GROUPS // BG
BM = 2000
NM = (N_USERS + N_ITEMS) // BM


def _user_msg_body(uh_ref, ue_ref, out_ref, emb_bf):
    @pl.when(pl.program_id(0) == 0)
    def _cache_embedding():
        emb_bf[...] = ue_ref[...].astype(jnp.bfloat16)

    out_ref[...] = jnp.dot(uh_ref[...].astype(jnp.bfloat16), emb_bf[...],
                           preferred_element_type=jnp.float32
                           ).astype(jnp.bfloat16)


def _linear_body(um_ref, im_ref, ge_ref, wt_ref, b_ref, msg_ref, msgbf_ref):
    ige = im_ref[...].astype(jnp.float32) * ge_ref[...]
    wt_b = wt_ref[...].astype(jnp.bfloat16)
    msg = jnp.dot(um_ref[...], wt_b[0:D, :],
                  preferred_element_type=jnp.float32)
    msg += jnp.dot(im_ref[...], wt_b[D:2 * D, :],
                   preferred_element_type=jnp.float32)
    msg += jnp.dot(ige.astype(jnp.bfloat16), wt_b[2 * D:3 * D, :],
                   preferred_element_type=jnp.float32)
    msg += b_ref[...]
    msg_ref[...] = msg
    msgbf_ref[...] = msg.astype(jnp.bfloat16)


def _agg_body(fh_ref, msgbf_ref, out_ref):
    out_ref[...] = jnp.dot(fh_ref[...].astype(jnp.bfloat16), msgbf_ref[...],
                           preferred_element_type=jnp.float32)


def _partial_msg(hyper, emb, n_cols):
    return pl.pallas_call(
        _user_msg_body,
        grid=(NG,),
        in_specs=[
            pl.BlockSpec((BG, n_cols), lambda g: (g, 0)),
            pl.BlockSpec((n_cols, D), lambda g: (0, 0)),
        ],
        out_specs=pl.BlockSpec((BG, D), lambda g: (g, 0)),
        out_shape=jax.ShapeDtypeStruct((N_GROUPS, D), jnp.bfloat16),
        scratch_shapes=[pltpu.VMEM((n_cols, D), jnp.bfloat16)],
        compiler_params=pltpu.CompilerParams(
            dimension_semantics=("arbitrary",)),
    )(hyper, emb)


def kernel(user_emb, item_emb, group_emb, user_hyper_graph, item_hyper_graph,
           full_hyper, W, b):
    wt = W.T
    b2 = b.reshape(1, D)

    um = _partial_msg(user_hyper_graph, user_emb, N_USERS)
    im = _partial_msg(item_hyper_graph, item_emb, N_ITEMS)

    msg, msg_bf = pl.pallas_call(
        _linear_body,
        grid=(1,),
        in_specs=[
            pl.BlockSpec((N_GROUPS, D), lambda m: (0, 0)),
            pl.BlockSpec((N_GROUPS, D), lambda m: (0, 0)),
            pl.BlockSpec((N_GROUPS, D), lambda m: (0, 0)),
            pl.BlockSpec((3 * D, D), lambda m: (0, 0)),
            pl.BlockSpec((1, D), lambda m: (0, 0)),
        ],
        out_specs=[
            pl.BlockSpec((N_GROUPS, D), lambda m: (0, 0)),
            pl.BlockSpec((N_GROUPS, D), lambda m: (0, 0)),
        ],
        out_shape=[
            jax.ShapeDtypeStruct((N_GROUPS, D), jnp.float32),
            jax.ShapeDtypeStruct((N_GROUPS, D), jnp.bfloat16),
        ],
        compiler_params=pltpu.CompilerParams(
            dimension_semantics=("arbitrary",)),
    )(um, im, group_emb, wt, b2)

    norm_emb = pl.pallas_call(
        _agg_body,
        grid=(NM,),
        in_specs=[
            pl.BlockSpec((BM, N_GROUPS), lambda m: (m, 0)),
            pl.BlockSpec((N_GROUPS, D), lambda m: (0, 0)),
        ],
        out_specs=pl.BlockSpec((BM, D), lambda m: (m, 0)),
        out_shape=jax.ShapeDtypeStruct((N_USERS + N_ITEMS, D), jnp.float32),
        compiler_params=pltpu.CompilerParams(
            dimension_semantics=("arbitrary",)),
    )(full_hyper, msg_bf)

    return (norm_emb, msg)


# speedup vs baseline: 1.0306x; 1.0306x over previous
import jax
import jax.numpy as jnp
from jax.experimental import pallas as pl
from jax.experimental.pallas import tpu as pltpu

N_USERS = 10000
N_ITEMS = 10000
N_GROUPS = 2048
D = 512

BK = 768
NK = (N_USERS + BK - 1) // BK
BM = 2000
NM = (N_USERS + N_ITEMS) // BM


def _msg_body(uh_ref, ih_ref, ue_ref, ie_ref, ge_ref, wt_ref, b_ref,
              msg_ref, acc):
    k = pl.program_id(0)

    def _partials(mask_cols):
        u_blk = uh_ref[...]
        i_blk = ih_ref[...]
        ue_blk = ue_ref[...]
        ie_blk = ie_ref[...]
        if mask_cols:
            col = k * BK + jax.lax.broadcasted_iota(jnp.int32, (N_GROUPS, BK), 1)
            row = k * BK + jax.lax.broadcasted_iota(jnp.int32, (BK, D), 0)
            u_blk = jnp.where(col < N_USERS, u_blk, 0.0)
            i_blk = jnp.where(col < N_ITEMS, i_blk, 0.0)
            ue_blk = jnp.where(row < N_USERS, ue_blk, 0.0)
            ie_blk = jnp.where(row < N_ITEMS, ie_blk, 0.0)
        pu = jnp.dot(u_blk.astype(jnp.bfloat16), ue_blk.astype(jnp.bfloat16),
                     preferred_element_type=jnp.float32)
        pi = jnp.dot(i_blk.astype(jnp.bfloat16), ie_blk.astype(jnp.bfloat16),
                     preferred_element_type=jnp.float32)
        return pu, pi

    @pl.when(k == 0)
    def _init():
        pu, pi = _partials(False)
        acc[:, 0:D] = pu
        acc[:, D:2 * D] = pi

    @pl.when((k != 0) & (k != NK - 1))
    def _accumulate():
        pu, pi = _partials(False)
        acc[:, 0:D] += pu
        acc[:, D:2 * D] += pi

    @pl.when(k == NK - 1)
    def _finalize():
        pu, pi = _partials(True)
        acc[:, 0:D] += pu
        acc[:, D:2 * D] += pi
        ui = acc[...]
        ige = ui[:, D:2 * D] * ge_ref[...]
        msg = jnp.dot(ui.astype(jnp.bfloat16), wt_ref[0:2 * D, :],
                      preferred_element_type=jnp.float32)
        msg += jnp.dot(ige.astype(jnp.bfloat16), wt_ref[2 * D:3 * D, :],
                       preferred_element_type=jnp.float32)
        msg_ref[...] = msg + b_ref[...]


def _agg_body(fh_ref, msg_ref, out_ref, msg_bf):
    @pl.when(pl.program_id(0) == 0)
    def _cache_msg():
        msg_bf[...] = msg_ref[...].astype(jnp.bfloat16)

    out_ref[...] = jnp.dot(fh_ref[...].astype(jnp.bfloat16), msg_bf[...],
                           preferred_element_type=jnp.float32)


def kernel(user_emb, item_emb, group_emb, user_hyper_graph, item_hyper_graph,
           full_hyper, W, b):
    wt = W.T
    b2 = b.reshape(1, D)

    msg = pl.pallas_call(
        _msg_body,
        grid=(NK,),
        in_specs=[
            pl.BlockSpec((N_GROUPS, BK), lambda k: (0, k)),
            pl.BlockSpec((N_GROUPS, BK), lambda k: (0, k)),
            pl.BlockSpec((BK, D), lambda k: (k, 0)),
            pl.BlockSpec((BK, D), lambda k: (k, 0)),
            pl.BlockSpec((N_GROUPS, D), lambda k: (0, 0)),
            pl.BlockSpec((3 * D, D), lambda k: (0, 0)),
            pl.BlockSpec((1, D), lambda k: (0, 0)),
        ],
        out_specs=pl.BlockSpec((N_GROUPS, D), lambda k: (0, 0)),
        out_shape=jax.ShapeDtypeStruct((N_GROUPS, D), jnp.float32),
        scratch_shapes=[pltpu.VMEM((N_GROUPS, 2 * D), jnp.float32)],
        compiler_params=pltpu.CompilerParams(
            dimension_semantics=("arbitrary",)),
    )(user_hyper_graph, item_hyper_graph, user_emb, item_emb, group_emb,
      wt, b2)

    norm_emb = pl.pallas_call(
        _agg_body,
        grid=(NM,),
        in_specs=[
            pl.BlockSpec((BM, N_GROUPS), lambda m: (m, 0)),
            pl.BlockSpec((N_GROUPS, D), lambda m: (0, 0)),
        ],
        out_specs=pl.BlockSpec((BM, D), lambda m: (m, 0)),
        out_shape=jax.ShapeDtypeStruct((N_USERS + N_ITEMS, D), jnp.float32),
        scratch_shapes=[pltpu.VMEM((N_GROUPS, D), jnp.bfloat16)],
        compiler_params=pltpu.CompilerParams(
            dimension_semantics=("arbitrary",)),
    )(full_hyper, msg)

    return (norm_emb, msg)
